# baseline (speedup 1.0000x reference)
import functools

import jax
import jax.numpy as jnp
from jax import lax
from jax.experimental import pallas as pl
from jax.experimental.pallas import tpu as pltpu


def _pack(w):
    co, ci, k, _ = w.shape
    return jnp.transpose(w, (0, 2, 3, 1)).reshape(co, k * k * ci)


def _fused_kernel(h, w, hw, k3, k5, c1, c3, c5, cr3,
                  x_ref, wred_ref, w1_ref, w3_ref, w5_ref, wp_ref,
                  bred_ref, b1_ref, b3_ref, b5_ref, bp_ref, o_ref):
    f32 = jnp.float32
    bf16 = jnp.bfloat16
    xf = x_ref[0]
    xb = xf.astype(bf16)

    pos = lax.broadcasted_iota(jnp.int32, (1, hw), 1)
    yy = pos // w
    xx = pos - yy * w

    def maskf(oy, ox):
        m = None
        if oy:
            m = (yy + oy >= 0) & (yy + oy < h)
        if ox:
            mc = (xx + ox >= 0) & (xx + ox < w)
            m = mc if m is None else m & mc
        if m is None:
            return None
        return jnp.where(m, f32(1.0), f32(0.0))

    def shift(a, s):
        if s == 0:
            return a
        k = s % hw
        return jnp.concatenate([a[:, k:], a[:, :k]], axis=-1)

    def conv(w_r, t, b_r):
        y = jnp.dot(w_r[...], t, preferred_element_type=f32)
        return jnp.maximum(y + b_r[...], 0.0)

    yred = conv(wred_ref, xb, bred_ref)
    r3x3 = yred[:cr3]
    r5x5 = yred[cr3:]

    def taps(r, k):
        p = (k - 1) // 2
        cols = []
        for oy in range(-p, p + 1):
            for ox in range(-p, p + 1):
                t = shift(r, oy * w + ox)
                mf = maskf(oy, ox)
                if mf is not None:
                    t = t * mf
                cols.append(t.astype(bf16))
        return jnp.concatenate(cols, axis=0)

    y3 = conv(w3_ref, taps(r3x3, k3), b3_ref)
    y5 = conv(w5_ref, taps(r5x5, k5), b5_ref)
    y1 = conv(w1_ref, xb, b1_ref)

    def masked_shift(a, s, mf):
        return shift(a, s) * mf + (mf - 1.0) * f32(1e30)

    hmax = xf
    for ox in (-1, 1):
        hmax = jnp.maximum(hmax, masked_shift(xf, ox, maskf(0, ox)))
    pooled = hmax
    for oy in (-1, 1):
        pooled = jnp.maximum(pooled, masked_shift(hmax, oy * w, maskf(oy, 0)))
    y4 = conv(wp_ref, pooled.astype(bf16), bp_ref)

    o_ref[0, 0:c1] = y1
    o_ref[0, c1:c1 + c3] = y3
    o_ref[0, c1 + c3:c1 + c3 + c5] = y5
    o_ref[0, c1 + c3 + c5:] = y4


@jax.jit
def kernel(x, w_b1, b_b1, w_b2a, b_b2a, w_b2b, b_b2b,
           w_b3a, b_b3a, w_b3b, b_b3b, w_b4, b_b4):
    n, cin, h, w = x.shape
    hw = h * w
    k3, k5 = w_b2b.shape[2], w_b3b.shape[2]
    c1, c3, c5, cp = w_b1.shape[0], w_b2b.shape[0], w_b3b.shape[0], w_b4.shape[0]
    cr3, cr5 = w_b2a.shape[0], w_b3a.shape[0]
    ct = c1 + c3 + c5 + cp

    x_ncm = x.astype(jnp.float32).reshape(n, cin, hw)

    bf16 = jnp.bfloat16
    wred = jnp.concatenate([_pack(w_b2a), _pack(w_b3a)], axis=0).astype(bf16)
    w1p = _pack(w_b1).astype(bf16)
    w3p = _pack(w_b2b).astype(bf16)
    w5p = _pack(w_b3b).astype(bf16)
    wpp = _pack(w_b4).astype(bf16)
    bred = jnp.concatenate([b_b2a, b_b3a]).reshape(-1, 1)
    b1r = b_b1.reshape(-1, 1)
    b3r = b_b2b.reshape(-1, 1)
    b5r = b_b3b.reshape(-1, 1)
    bpr = b_b4.reshape(-1, 1)

    kern = functools.partial(_fused_kernel, h, w, hw, k3, k5,
                             c1, c3, c5, cr3)
    out = pl.pallas_call(
        kern,
        out_shape=jax.ShapeDtypeStruct((n, ct, hw), jnp.float32),
        grid=(n,),
        in_specs=[
            pl.BlockSpec((1, cin, hw), lambda i: (i, 0, 0)),
            pl.BlockSpec((cr3 + cr5, cin), lambda i: (0, 0)),
            pl.BlockSpec((c1, cin), lambda i: (0, 0)),
            pl.BlockSpec((c3, k3 * k3 * cr3), lambda i: (0, 0)),
            pl.BlockSpec((c5, k5 * k5 * cr5), lambda i: (0, 0)),
            pl.BlockSpec((cp, cin), lambda i: (0, 0)),
            pl.BlockSpec((cr3 + cr5, 1), lambda i: (0, 0)),
            pl.BlockSpec((c1, 1), lambda i: (0, 0)),
            pl.BlockSpec((c3, 1), lambda i: (0, 0)),
            pl.BlockSpec((c5, 1), lambda i: (0, 0)),
            pl.BlockSpec((cp, 1), lambda i: (0, 0)),
        ],
        out_specs=pl.BlockSpec((1, ct, hw), lambda i: (i, 0, 0)),
        compiler_params=pltpu.CompilerParams(
            dimension_semantics=("parallel",),
            vmem_limit_bytes=32 * 1024 * 1024),
    )(x_ncm, wred, w1p, w3p, w5p, wpp, bred, b1r, b3r, b5r, bpr)
    return out.reshape(n, ct, h, w)


# speedup vs baseline: 1.0861x; 1.0861x over previous
import functools

import jax
import jax.numpy as jnp
from jax import lax
from jax.experimental import pallas as pl
from jax.experimental.pallas import tpu as pltpu


def _pack(w):
    co, ci, k, _ = w.shape
    return jnp.transpose(w, (0, 2, 3, 1)).reshape(co, k * k * ci)


def _fused_kernel(h, w, hw, hwp, k3, k5, c1, c3, c5, cr3,
                  x_ref, wred_ref, w1_ref, w3_ref, w5_ref, wp_ref,
                  bred_ref, b1_ref, b3_ref, b5_ref, bp_ref, o_ref):
    f32 = jnp.float32
    bf16 = jnp.bfloat16
    xb = x_ref[0]

    pos = lax.broadcasted_iota(jnp.int32, (1, hwp), 1)
    yy = pos // w
    xx = pos - yy * w
    in_img = pos < hw

    def maskf(oy, ox, pad_tail):
        m = in_img if pad_tail else None
        if oy:
            mr = (yy + oy >= 0) & (yy + oy < h)
            m = mr if m is None else m & mr
        if ox:
            mc = (xx + ox >= 0) & (xx + ox < w)
            m = mc if m is None else m & mc
        if m is None:
            return None
        return jnp.where(m, f32(1.0), f32(0.0))

    def shift(a, s):
        if s == 0:
            return a
        k = s % hwp
        return jnp.concatenate([a[:, k:], a[:, :k]], axis=-1)

    def conv(w_r, t, b_r):
        y = jnp.dot(w_r[...], t, preferred_element_type=f32)
        return jnp.maximum(y + b_r[...], 0.0)

    yred = conv(wred_ref, xb, bred_ref)
    r3x3 = yred[:cr3]
    r5x5 = yred[cr3:]

    def taps(r, k):
        p = (k - 1) // 2
        cols = []
        for oy in range(-p, p + 1):
            for ox in range(-p, p + 1):
                t = shift(r, oy * w + ox) * maskf(oy, ox, True)
                cols.append(t.astype(bf16))
        return jnp.concatenate(cols, axis=0)

    y3 = conv(w3_ref, taps(r3x3, k3), b3_ref)
    y5 = conv(w5_ref, taps(r5x5, k5), b5_ref)
    y1 = conv(w1_ref, xb, b1_ref)

    xf = xb.astype(f32)
    def masked_shift(a, s, mf):
        return shift(a, s) * mf + (mf - 1.0) * f32(1e30)

    hmax = xf
    for ox in (-1, 1):
        hmax = jnp.maximum(hmax, masked_shift(xf, ox, maskf(0, ox, False)))
    pooled = hmax
    for oy in (-1, 1):
        pooled = jnp.maximum(pooled, masked_shift(hmax, oy * w, maskf(oy, 0, False)))
    y4 = conv(wp_ref, pooled.astype(bf16), bp_ref)

    o_ref[0, 0:c1] = y1
    o_ref[0, c1:c1 + c3] = y3
    o_ref[0, c1 + c3:c1 + c3 + c5] = y5
    o_ref[0, c1 + c3 + c5:] = y4


@jax.jit
def kernel(x, w_b1, b_b1, w_b2a, b_b2a, w_b2b, b_b2b,
           w_b3a, b_b3a, w_b3b, b_b3b, w_b4, b_b4):
    n, cin, h, w = x.shape
    hw = h * w
    hwp = (hw + 127) // 128 * 128
    k3, k5 = w_b2b.shape[2], w_b3b.shape[2]
    c1, c3, c5, cp = w_b1.shape[0], w_b2b.shape[0], w_b3b.shape[0], w_b4.shape[0]
    cr3, cr5 = w_b2a.shape[0], w_b3a.shape[0]
    ct = c1 + c3 + c5 + cp

    bf16 = jnp.bfloat16
    xr = x.astype(bf16).reshape(n, cin, hw)
    x_ncm = jnp.pad(xr, ((0, 0), (0, 0), (0, hwp - hw)))

    wred = jnp.concatenate([_pack(w_b2a), _pack(w_b3a)], axis=0).astype(bf16)
    w1p = _pack(w_b1).astype(bf16)
    w3p = _pack(w_b2b).astype(bf16)
    w5p = _pack(w_b3b).astype(bf16)
    wpp = _pack(w_b4).astype(bf16)
    bred = jnp.concatenate([b_b2a, b_b3a]).reshape(-1, 1)
    b1r = b_b1.reshape(-1, 1)
    b3r = b_b2b.reshape(-1, 1)
    b5r = b_b3b.reshape(-1, 1)
    bpr = b_b4.reshape(-1, 1)

    kern = functools.partial(_fused_kernel, h, w, hw, hwp, k3, k5,
                             c1, c3, c5, cr3)
    img = lambda i: (i, 0, 0)
    rep = lambda i: (0, 0)
    out = pl.pallas_call(
        kern,
        out_shape=jax.ShapeDtypeStruct((n, ct, hwp), jnp.float32),
        grid=(n,),
        in_specs=[
            pl.BlockSpec((1, cin, hwp), img),
            pl.BlockSpec((cr3 + cr5, cin), rep),
            pl.BlockSpec((c1, cin), rep),
            pl.BlockSpec((c3, k3 * k3 * cr3), rep),
            pl.BlockSpec((c5, k5 * k5 * cr5), rep),
            pl.BlockSpec((cp, cin), rep),
            pl.BlockSpec((cr3 + cr5, 1), rep),
            pl.BlockSpec((c1, 1), rep),
            pl.BlockSpec((c3, 1), rep),
            pl.BlockSpec((c5, 1), rep),
            pl.BlockSpec((cp, 1), rep),
        ],
        out_specs=pl.BlockSpec((1, ct, hwp), img),
        compiler_params=pltpu.CompilerParams(
            dimension_semantics=("parallel",),
            vmem_limit_bytes=32 * 1024 * 1024),
    )(x_ncm, wred, w1p, w3p, w5p, wpp, bred, b1r, b3r, b5r, bpr)
    return out[:, :, :hw].reshape(n, ct, h, w)


# speedup vs baseline: 1.4263x; 1.3132x over previous
import functools

import jax
import jax.numpy as jnp
from jax import lax
from jax.experimental import pallas as pl
from jax.experimental.pallas import tpu as pltpu


def _pack(w):
    co, ci, k, _ = w.shape
    return jnp.transpose(w, (0, 2, 3, 1)).reshape(co, k * k * ci)


def _fused_kernel(h, w, hw, hwp, k3, k5, c1, c3, c5, cr3,
                  x_ref, wred_ref, w1_ref, w3_ref, w5_ref, wp_ref,
                  bred_ref, b1_ref, b3_ref, b5_ref, bp_ref, o_ref):
    f32 = jnp.float32
    bf16 = jnp.bfloat16
    i32 = jnp.int32
    xb = x_ref[0]

    pos = lax.broadcasted_iota(i32, (1, hwp), 1)
    yy = pos // w
    xx = pos - yy * w
    one = jnp.asarray(1.0, bf16)
    in_img_bf = jnp.where(pos < hw, f32(1.0), f32(0.0)).astype(bf16)

    def colmask(ox):
        m = (xx + ox >= 0) & (xx + ox < w)
        return jnp.where(m, f32(1.0), f32(0.0)).astype(bf16)

    def rowmask(oy):
        m = (yy + oy >= 0) & (yy + oy < h)
        return jnp.where(m, f32(1.0), f32(0.0)).astype(bf16)

    def shift_i(a, s):
        if s == 0:
            return a
        k = s % hwp
        return jnp.concatenate([a[:, k:], a[:, :k]], axis=-1)

    def shift_bf(a_bf, s):
        return pltpu.bitcast(shift_i(pltpu.bitcast(a_bf, i32), s), bf16)

    def conv(w_r, t, b_r):
        y = jnp.dot(w_r[...], t, preferred_element_type=f32)
        return jnp.maximum(y + b_r[...], 0.0)

    yred = conv(wred_ref, xb, bred_ref)
    rb = yred.astype(bf16) * in_img_bf
    r3b = rb[:cr3]
    r5b = rb[cr3:]

    def taps(r, k):
        p = (k - 1) // 2
        cols = []
        for oy in range(-p, p + 1):
            for ox in range(-p, p + 1):
                t = shift_bf(r, oy * w + ox)
                if ox:
                    t = t * colmask(ox)
                cols.append(t)
        return jnp.concatenate(cols, axis=0)

    y3 = conv(w3_ref, taps(r3b, k3), b3_ref)
    y5 = conv(w5_ref, taps(r5b, k5), b5_ref)
    y1 = conv(w1_ref, xb, b1_ref)

    big = jnp.asarray(1e30, bf16)
    def masked_shift(a, s, mf):
        return shift_bf(a, s) * mf + (mf - one) * big

    hmax = xb
    for ox in (-1, 1):
        hmax = jnp.maximum(hmax, masked_shift(xb, ox, colmask(ox)))
    pooled = hmax
    for oy in (-1, 1):
        pooled = jnp.maximum(pooled, masked_shift(hmax, oy * w, rowmask(oy)))
    y4 = conv(wp_ref, pooled, bp_ref)

    o_ref[0, 0:c1] = y1
    o_ref[0, c1:c1 + c3] = y3
    o_ref[0, c1 + c3:c1 + c3 + c5] = y5
    o_ref[0, c1 + c3 + c5:] = y4


@jax.jit
def kernel(x, w_b1, b_b1, w_b2a, b_b2a, w_b2b, b_b2b,
           w_b3a, b_b3a, w_b3b, b_b3b, w_b4, b_b4):
    n, cin, h, w = x.shape
    hw = h * w
    hwp = (hw + 127) // 128 * 128
    k3, k5 = w_b2b.shape[2], w_b3b.shape[2]
    c1, c3, c5, cp = w_b1.shape[0], w_b2b.shape[0], w_b3b.shape[0], w_b4.shape[0]
    cr3, cr5 = w_b2a.shape[0], w_b3a.shape[0]
    ct = c1 + c3 + c5 + cp

    bf16 = jnp.bfloat16
    xr = x.astype(bf16).reshape(n, cin, hw)
    x_ncm = jnp.pad(xr, ((0, 0), (0, 0), (0, hwp - hw)))

    wred = jnp.concatenate([_pack(w_b2a), _pack(w_b3a)], axis=0).astype(bf16)
    w1p = _pack(w_b1).astype(bf16)
    w3p = _pack(w_b2b).astype(bf16)
    w5p = _pack(w_b3b).astype(bf16)
    wpp = _pack(w_b4).astype(bf16)
    bred = jnp.concatenate([b_b2a, b_b3a]).reshape(-1, 1)
    b1r = b_b1.reshape(-1, 1)
    b3r = b_b2b.reshape(-1, 1)
    b5r = b_b3b.reshape(-1, 1)
    bpr = b_b4.reshape(-1, 1)

    kern = functools.partial(_fused_kernel, h, w, hw, hwp, k3, k5,
                             c1, c3, c5, cr3)
    img = lambda i: (i, 0, 0)
    rep = lambda i: (0, 0)
    out = pl.pallas_call(
        kern,
        out_shape=jax.ShapeDtypeStruct((n, ct, hwp), jnp.float32),
        grid=(n,),
        in_specs=[
            pl.BlockSpec((1, cin, hwp), img),
            pl.BlockSpec((cr3 + cr5, cin), rep),
            pl.BlockSpec((c1, cin), rep),
            pl.BlockSpec((c3, k3 * k3 * cr3), rep),
            pl.BlockSpec((c5, k5 * k5 * cr5), rep),
            pl.BlockSpec((cp, cin), rep),
            pl.BlockSpec((cr3 + cr5, 1), rep),
            pl.BlockSpec((c1, 1), rep),
            pl.BlockSpec((c3, 1), rep),
            pl.BlockSpec((c5, 1), rep),
            pl.BlockSpec((cp, 1), rep),
        ],
        out_specs=pl.BlockSpec((1, ct, hwp), img),
        compiler_params=pltpu.CompilerParams(
            dimension_semantics=("parallel",),
            vmem_limit_bytes=32 * 1024 * 1024),
    )(x_ncm, wred, w1p, w3p, w5p, wpp, bred, b1r, b3r, b5r, bpr)
    return out[:, :, :hw].reshape(n, ct, h, w)


# speedup vs baseline: 1.5287x; 1.0718x over previous
import functools

import jax
import jax.numpy as jnp
from jax import lax
from jax.experimental import pallas as pl
from jax.experimental.pallas import tpu as pltpu


def _pack(w):
    co, ci, k, _ = w.shape
    return jnp.transpose(w, (0, 2, 3, 1)).reshape(co, k * k * ci)


def _fused_kernel(h, w, hw, hwp, k3, k5, c1, c3, c5, cr3,
                  x_ref, wred_ref, w1_ref, w3_ref, w5_ref, wp_ref,
                  bred_ref, b1_ref, b3_ref, b5_ref, bp_ref, o_ref):
    f32 = jnp.float32
    bf16 = jnp.bfloat16
    i32 = jnp.int32
    xb = x_ref[0]

    pos = lax.broadcasted_iota(i32, (1, hwp), 1)
    yy = pos // w
    xx = pos - yy * w
    one = jnp.asarray(1.0, bf16)
    in_img_bf = jnp.where(pos < hw, f32(1.0), f32(0.0)).astype(bf16)

    def colmask(ox):
        m = (xx + ox >= 0) & (xx + ox < w)
        return jnp.where(m, f32(1.0), f32(0.0)).astype(bf16)

    def rowmask(oy):
        m = (yy + oy >= 0) & (yy + oy < h)
        return jnp.where(m, f32(1.0), f32(0.0)).astype(bf16)

    def shift_i(a, s):
        if s == 0:
            return a
        k = s % hwp
        return jnp.concatenate([a[:, k:], a[:, :k]], axis=-1)

    def shift_bf(a_bf, s):
        return pltpu.bitcast(shift_i(pltpu.bitcast(a_bf, i32), s), bf16)

    def conv(w_r, t, b_r):
        y = jnp.dot(w_r[...], t, preferred_element_type=f32)
        return jnp.maximum(y + b_r[...], 0.0)

    yred = conv(wred_ref, xb, bred_ref)
    rb = yred.astype(bf16) * in_img_bf
    r3b = rb[:cr3]
    r5b = rb[cr3:]

    def taps(r, k):
        p = (k - 1) // 2
        cols = []
        for oy in range(-p, p + 1):
            for ox in range(-p, p + 1):
                t = shift_bf(r, oy * w + ox)
                if ox:
                    t = t * colmask(ox)
                cols.append(t)
        return jnp.concatenate(cols, axis=0)

    y3 = conv(w3_ref, taps(r3b, k3), b3_ref)
    y5 = conv(w5_ref, taps(r5b, k5), b5_ref)
    y1 = conv(w1_ref, xb, b1_ref)

    big = jnp.asarray(1e30, bf16)
    def masked_shift(a, s, mf):
        return shift_bf(a, s) * mf + (mf - one) * big

    hmax = xb
    for ox in (-1, 1):
        hmax = jnp.maximum(hmax, masked_shift(xb, ox, colmask(ox)))
    pooled = hmax
    for oy in (-1, 1):
        pooled = jnp.maximum(pooled, masked_shift(hmax, oy * w, rowmask(oy)))
    y4 = conv(wp_ref, pooled, bp_ref)

    o_ref[0, 0:c1] = y1
    o_ref[0, c1:c1 + c3] = y3
    o_ref[0, c1 + c3:c1 + c3 + c5] = y5
    o_ref[0, c1 + c3 + c5:] = y4


@jax.jit
def kernel(x, w_b1, b_b1, w_b2a, b_b2a, w_b2b, b_b2b,
           w_b3a, b_b3a, w_b3b, b_b3b, w_b4, b_b4):
    n, cin, h, w = x.shape
    hw = h * w
    hwp = (hw + 127) // 128 * 128
    k3, k5 = w_b2b.shape[2], w_b3b.shape[2]
    c1, c3, c5, cp = w_b1.shape[0], w_b2b.shape[0], w_b3b.shape[0], w_b4.shape[0]
    cr3, cr5 = w_b2a.shape[0], w_b3a.shape[0]
    ct = c1 + c3 + c5 + cp

    bf16 = jnp.bfloat16
    xr = x.astype(bf16).reshape(n, cin, hw)
    x_ncm = jnp.pad(xr, ((0, 0), (0, 0), (0, hwp - hw)))

    wred = jnp.concatenate([_pack(w_b2a), _pack(w_b3a)], axis=0).astype(bf16)
    w1p = _pack(w_b1).astype(bf16)
    w3p = _pack(w_b2b).astype(bf16)
    w5p = _pack(w_b3b).astype(bf16)
    wpp = _pack(w_b4).astype(bf16)
    bred = jnp.concatenate([b_b2a, b_b3a]).reshape(-1, 1)
    b1r = b_b1.reshape(-1, 1)
    b3r = b_b2b.reshape(-1, 1)
    b5r = b_b3b.reshape(-1, 1)
    bpr = b_b4.reshape(-1, 1)

    kern = functools.partial(_fused_kernel, h, w, hw, hwp, k3, k5,
                             c1, c3, c5, cr3)
    img = lambda i: (i, 0, 0)
    rep = lambda i: (0, 0)
    out = pl.pallas_call(
        kern,
        out_shape=jax.ShapeDtypeStruct((n, ct, hwp), jnp.float32),
        grid=(n,),
        in_specs=[
            pl.BlockSpec((1, cin, hwp), img),
            pl.BlockSpec((cr3 + cr5, cin), rep),
            pl.BlockSpec((c1, cin), rep),
            pl.BlockSpec((c3, k3 * k3 * cr3), rep),
            pl.BlockSpec((c5, k5 * k5 * cr5), rep),
            pl.BlockSpec((cp, cin), rep),
            pl.BlockSpec((cr3 + cr5, 1), rep),
            pl.BlockSpec((c1, 1), rep),
            pl.BlockSpec((c3, 1), rep),
            pl.BlockSpec((c5, 1), rep),
            pl.BlockSpec((cp, 1), rep),
        ],
        out_specs=pl.BlockSpec((1, ct, hwp), img),
        compiler_params=pltpu.CompilerParams(
            dimension_semantics=("parallel",),
            allow_input_fusion=[True] + [False] * 10,
            vmem_limit_bytes=32 * 1024 * 1024),
    )(x_ncm, wred, w1p, w3p, w5p, wpp, bred, b1r, b3r, b5r, bpr)
    return out[:, :, :hw].reshape(n, ct, h, w)


# speedup vs baseline: 1.5296x; 1.0006x over previous
import functools

import jax
import jax.numpy as jnp
from jax import lax
from jax.experimental import pallas as pl
from jax.experimental.pallas import tpu as pltpu


def _pack(w):
    co, ci, k, _ = w.shape
    return jnp.transpose(w, (0, 2, 3, 1)).reshape(co, k * k * ci)


def _fused_kernel(h, w, hw, hwp, k3, k5, c1, c3, c5, cr3, gb,
                  x_ref, wred_ref, w1_ref, w3_ref, w5_ref, wp_ref,
                  bred_ref, b1_ref, b3_ref, b5_ref, bp_ref, o_ref):
    f32 = jnp.float32
    bf16 = jnp.bfloat16
    i32 = jnp.int32

    pos = lax.broadcasted_iota(i32, (1, hwp), 1)
    yy = pos // w
    xx = pos - yy * w
    one = jnp.asarray(1.0, bf16)
    in_img_bf = jnp.where(pos < hw, f32(1.0), f32(0.0)).astype(bf16)

    def colmask(ox):
        m = (xx + ox >= 0) & (xx + ox < w)
        return jnp.where(m, f32(1.0), f32(0.0)).astype(bf16)

    def rowmask(oy):
        m = (yy + oy >= 0) & (yy + oy < h)
        return jnp.where(m, f32(1.0), f32(0.0)).astype(bf16)

    def shift_i(a, s):
        if s == 0:
            return a
        k = s % hwp
        return jnp.concatenate([a[:, k:], a[:, :k]], axis=-1)

    def shift_bf(a_bf, s):
        return pltpu.bitcast(shift_i(pltpu.bitcast(a_bf, i32), s), bf16)

    def conv(w_r, t, b_r):
        y = jnp.dot(w_r[...], t, preferred_element_type=f32)
        return jnp.maximum(y + b_r[...], 0.0)

    big = jnp.asarray(1e30, bf16)

    def one_image(j):
        xb = x_ref[j]

        yred = conv(wred_ref, xb, bred_ref)
        rb = yred.astype(bf16) * in_img_bf
        r3b = rb[:cr3]
        r5b = rb[cr3:]

        def taps(r, k):
            p = (k - 1) // 2
            cols = []
            for oy in range(-p, p + 1):
                for ox in range(-p, p + 1):
                    t = shift_bf(r, oy * w + ox)
                    if ox:
                        t = t * colmask(ox)
                    cols.append(t)
            return jnp.concatenate(cols, axis=0)

        y3 = conv(w3_ref, taps(r3b, k3), b3_ref)
        y5 = conv(w5_ref, taps(r5b, k5), b5_ref)
        y1 = conv(w1_ref, xb, b1_ref)

        def masked_shift(a, s, mf):
            return shift_bf(a, s) * mf + (mf - one) * big

        hmax = xb
        for ox in (-1, 1):
            hmax = jnp.maximum(hmax, masked_shift(xb, ox, colmask(ox)))
        pooled = hmax
        for oy in (-1, 1):
            pooled = jnp.maximum(pooled, masked_shift(hmax, oy * w, rowmask(oy)))
        y4 = conv(wp_ref, pooled, bp_ref)

        o_ref[j, 0:c1] = y1
        o_ref[j, c1:c1 + c3] = y3
        o_ref[j, c1 + c3:c1 + c3 + c5] = y5
        o_ref[j, c1 + c3 + c5:] = y4

    for j in range(gb):
        one_image(j)


@jax.jit
def kernel(x, w_b1, b_b1, w_b2a, b_b2a, w_b2b, b_b2b,
           w_b3a, b_b3a, w_b3b, b_b3b, w_b4, b_b4):
    n, cin, h, w = x.shape
    hw = h * w
    hwp = (hw + 127) // 128 * 128
    k3, k5 = w_b2b.shape[2], w_b3b.shape[2]
    c1, c3, c5, cp = w_b1.shape[0], w_b2b.shape[0], w_b3b.shape[0], w_b4.shape[0]
    cr3, cr5 = w_b2a.shape[0], w_b3a.shape[0]
    ct = c1 + c3 + c5 + cp

    bf16 = jnp.bfloat16
    xr = x.astype(bf16).reshape(n, cin, hw)
    x_ncm = jnp.pad(xr, ((0, 0), (0, 0), (0, hwp - hw)))

    wred = jnp.concatenate([_pack(w_b2a), _pack(w_b3a)], axis=0).astype(bf16)
    w1p = _pack(w_b1).astype(bf16)
    w3p = _pack(w_b2b).astype(bf16)
    w5p = _pack(w_b3b).astype(bf16)
    wpp = _pack(w_b4).astype(bf16)
    bred = jnp.concatenate([b_b2a, b_b3a]).reshape(-1, 1)
    b1r = b_b1.reshape(-1, 1)
    b3r = b_b2b.reshape(-1, 1)
    b5r = b_b3b.reshape(-1, 1)
    bpr = b_b4.reshape(-1, 1)

    gb = 2 if n % 2 == 0 else 1
    kern = functools.partial(_fused_kernel, h, w, hw, hwp, k3, k5,
                             c1, c3, c5, cr3, gb)
    img = lambda i: (i, 0, 0)
    rep = lambda i: (0, 0)
    out = pl.pallas_call(
        kern,
        out_shape=jax.ShapeDtypeStruct((n, ct, hwp), jnp.float32),
        grid=(n // gb,),
        in_specs=[
            pl.BlockSpec((gb, cin, hwp), img),
            pl.BlockSpec((cr3 + cr5, cin), rep),
            pl.BlockSpec((c1, cin), rep),
            pl.BlockSpec((c3, k3 * k3 * cr3), rep),
            pl.BlockSpec((c5, k5 * k5 * cr5), rep),
            pl.BlockSpec((cp, cin), rep),
            pl.BlockSpec((cr3 + cr5, 1), rep),
            pl.BlockSpec((c1, 1), rep),
            pl.BlockSpec((c3, 1), rep),
            pl.BlockSpec((c5, 1), rep),
            pl.BlockSpec((cp, 1), rep),
        ],
        out_specs=pl.BlockSpec((gb, ct, hwp), img),
        compiler_params=pltpu.CompilerParams(
            dimension_semantics=("parallel",),
            allow_input_fusion=[True] + [False] * 10,
            vmem_limit_bytes=32 * 1024 * 1024),
    )(x_ncm, wred, w1p, w3p, w5p, wpp, bred, b1r, b3r, b5r, bpr)
    return out[:, :, :hw].reshape(n, ct, h, w)
